# baseline (speedup 1.0000x reference)
import functools

import jax
import jax.numpy as jnp
from jax import lax
from jax.experimental import pallas as pl
from jax.experimental.pallas import tpu as pltpu
from jax.experimental.pallas import tpu_sc as plsc

N = 100000
M = 50000
D = 128

NC = 2
NS = 16
L = 16
NW = NC * NS
RANGE = 3200
N_PAD = NW * RANGE
CH = 2000
GCH = 128
NCG = RANGE // GCH
BLK = 5000
Z = 4096
ZBASE = 2 * M

_mesh = plsc.VectorSubcoreMesh(
    core_axis_name="c", subcore_axis_name="s", num_cores=NC, num_subcores=NS
)
_sc_params = pltpu.CompilerParams(
    needs_layout_passes=False, use_tc_tiling_on_sc=True)


@functools.partial(
    pl.kernel,
    out_type=(
        jax.ShapeDtypeStruct((NW, NCG, GCH), jnp.int32),
        jax.ShapeDtypeStruct((NW, NCG, GCH), jnp.int32),
    ),
    mesh=_mesh,
    compiler_params=_sc_params,
    scratch_types=[
        pltpu.VMEM((CH,), jnp.int32),
        pltpu.VMEM((CH,), jnp.int32),
        pltpu.VMEM((RANGE,), jnp.int32),
        pltpu.VMEM((NCG, GCH), jnp.int32),
        pltpu.SemaphoreType.DMA,
        pltpu.SemaphoreType.DMA,
    ],
)
def _sc_scan(o_idx, s_idx, gox, gsx, idx_a, idx_b, slot_v, gidx_v,
             isem_a, isem_b):
    cid = lax.axis_index("c")
    sid = lax.axis_index("s")
    wid = sid * NC + cid
    base = wid * RANGE
    lane = lax.iota(jnp.int32, L)

    GRP = 5
    NGRP = CH // L // GRP
    NCHUNK = M // CH

    def scan_stream(idx_hbm, g_hbm, soff):
        neg = jnp.full((L,), -1, jnp.int32)

        def init_body(v, carry):
            slot_v[pl.ds(v * L, L)] = neg
            return carry

        lax.fori_loop(0, RANGE // L, init_body, 0)

        pltpu.async_copy(idx_hbm.at[pl.ds(0, CH)], idx_a, isem_a)

        def process_half(c, idx_v):
            def grp_body(g, carry2):
                j0 = c * (CH // L) + g * GRP
                lostv = jnp.zeros((L,), jnp.int32)
                datas = []
                for u in range(GRP):
                    idxv = idx_v[pl.ds((g * GRP + u) * L, L)]
                    loc = idxv - base
                    m = (loc >= 0) & (loc < RANGE)
                    locm = jnp.where(m, loc, 0)
                    iv = (j0 + u) * L + lane
                    plsc.store_scatter(slot_v, (locm,), iv, mask=m)
                    datas.append((locm, iv, m))
                for locm, iv, m in datas:
                    back = plsc.load_gather(slot_v, (locm,), mask=m)
                    lostv = lostv | jnp.where(m & (back != iv), 1, 0)

                @pl.when(jnp.any(lostv > 0))
                def _repair():
                    def rep_body(u2, carry3):
                        idxv = idx_v[pl.ds((g * GRP + u2) * L, L)]
                        loc = idxv - base
                        m = (loc >= 0) & (loc < RANGE)
                        locm = jnp.where(m, loc, 0)
                        iv = (j0 + u2) * L + lane
                        for l in range(L):
                            plsc.store_scatter(
                                slot_v, (locm,), iv, mask=m & (lane == l))
                        return carry3

                    lax.fori_loop(0, GRP, rep_body, 0)

                return carry2

            lax.fori_loop(0, NGRP, grp_body, 0)

        def chunk_body(c, carry):
            @pl.when(c % 2 == 0)
            def _even():
                pltpu.make_async_copy(
                    idx_hbm.at[pl.ds(c * CH, CH)], idx_a, isem_a).wait()

                @pl.when(c + 1 < NCHUNK)
                def _pf():
                    pltpu.async_copy(
                        idx_hbm.at[pl.ds((c + 1) * CH, CH)], idx_b, isem_b)

                process_half(c, idx_a)

            @pl.when(c % 2 == 1)
            def _odd():
                pltpu.make_async_copy(
                    idx_hbm.at[pl.ds(c * CH, CH)], idx_b, isem_b).wait()

                @pl.when(c + 1 < NCHUNK)
                def _pf2():
                    pltpu.async_copy(
                        idx_hbm.at[pl.ds((c + 1) * CH, CH)], idx_a, isem_a)

                process_half(c, idx_b)

            return carry

        lax.fori_loop(0, NCHUNK, chunk_body, 0)

        def gidx_body(v, carry):
            sl = slot_v[pl.ds(v * L, L)]
            inval = sl < 0
            zfill = ZBASE + ((base + v * L + lane) & (Z - 1))
            g = jnp.where(inval, zfill, sl + soff)
            gidx_v[v // 8, pl.ds((v % 8) * L, L)] = g
            return carry

        lax.fori_loop(0, RANGE // L, gidx_body, 0)
        pltpu.sync_copy(gidx_v, g_hbm.at[wid])

    scan_stream(o_idx, gox, 0)
    scan_stream(s_idx, gsx, M)


@functools.partial(
    pl.kernel,
    out_type=jax.ShapeDtypeStruct((N_PAD, D), jnp.float32),
    mesh=_mesh,
    compiler_params=_sc_params,
    scratch_types=[
        pltpu.VMEM((NCG, GCH), jnp.int32),
        pltpu.VMEM((NCG, GCH), jnp.int32),
        pltpu.VMEM((GCH, D), jnp.float32),
        pltpu.VMEM((GCH, D), jnp.float32),
        pltpu.VMEM((GCH, D), jnp.float32),
        pltpu.VMEM((GCH, D), jnp.float32),
        pltpu.VMEM((GCH, D), jnp.float32),
        pltpu.VMEM((GCH, D), jnp.float32),
        pltpu.SemaphoreType.DMA,
        pltpu.SemaphoreType.DMA,
        pltpu.SemaphoreType.DMA,
        pltpu.SemaphoreType.DMA,
        pltpu.SemaphoreType.DMA,
        pltpu.SemaphoreType.DMA,
        pltpu.SemaphoreType.DMA,
    ],
)
def _sc_gather(big, gox, gsx, susum,
               gidx_o, gidx_s, bo0, bo1, bs0, bs1, bu0, bu1,
               gsem_o0, gsem_o1, gsem_s0, gsem_s1, osem0, osem1, lsem):
    cid = lax.axis_index("c")
    sid = lax.axis_index("s")
    wid = sid * NC + cid
    base = wid * RANGE

    pltpu.async_copy(gox.at[wid], gidx_o, lsem).wait()
    pltpu.async_copy(gsx.at[wid], gidx_s, lsem).wait()

    obufs = (bo0, bo1)
    sbufs = (bs0, bs1)
    ubufs = (bu0, bu1)
    osems = (gsem_o0, gsem_o1)
    ssems = (gsem_s0, gsem_s1)
    usems = (osem0, osem1)

    in_o = [None] * NCG
    in_s = [None] * NCG
    outd = [None] * NCG

    def fire(c):
        b = c % 2
        in_o[c] = pltpu.async_copy(big.at[gidx_o.at[c]], obufs[b], osems[b])
        in_s[c] = pltpu.async_copy(big.at[gidx_s.at[c]], sbufs[b], ssems[b])

    def addsum(b):
        def ab(k, carry):
            bu = ubufs[b]
            bo = obufs[b]
            bs = sbufs[b]
            for u in range(D // L):
                cs = pl.ds(u * L, L)
                bu[k, cs] = bo[k, cs] + bs[k, cs]
            return carry

        lax.fori_loop(0, GCH, ab, 0)

    fire(0)
    for c in range(NCG):
        b = c % 2
        if c + 1 < NCG:
            fire(c + 1)
        in_o[c].wait()
        in_s[c].wait()
        if c >= 2:
            outd[c - 2].wait()
        addsum(b)
        outd[c] = pltpu.async_copy(
            ubufs[b], susum.at[pl.ds(base + c * GCH, GCH)], usems[b])
    outd[NCG - 2].wait()
    outd[NCG - 1].wait()


def _tc_body(u_ref, tar_ref, w1_ref, b1_ref, w2_ref, b2_ref, out_ref):
    su = 0.5 * u_ref[...]
    h1 = jnp.maximum(
        jnp.dot(su, w1_ref[...], preferred_element_type=jnp.float32)
        + b1_ref[...], 0.0)
    h2 = jnp.maximum(
        jnp.dot(tar_ref[...], w2_ref[...], preferred_element_type=jnp.float32)
        + b2_ref[...], 0.0)
    out_ref[...] = su + h1 + h2


_tc_apply = pl.pallas_call(
    _tc_body,
    grid=(N // BLK,),
    in_specs=[
        pl.BlockSpec((BLK, D), lambda i: (i, 0)),
        pl.BlockSpec((BLK, D), lambda i: (i, 0)),
        pl.BlockSpec((D, D), lambda i: (0, 0)),
        pl.BlockSpec((1, D), lambda i: (0, 0)),
        pl.BlockSpec((D, D), lambda i: (0, 0)),
        pl.BlockSpec((1, D), lambda i: (0, 0)),
    ],
    out_specs=pl.BlockSpec((BLK, D), lambda i: (i, 0)),
    out_shape=jax.ShapeDtypeStruct((N, D), jnp.float32),
)


def kernel(msg_from_o, msg_from_s, o_ava_idx, s_ava_idx, tar_feat,
           W1, b1, W2, b2):
    big = jnp.concatenate(
        [msg_from_o, msg_from_s, jnp.zeros((Z, D), jnp.float32)], axis=0)
    gox, gsx = _sc_scan(
        o_ava_idx.astype(jnp.int32), s_ava_idx.astype(jnp.int32))
    susum = _sc_gather(big, gox, gsx)
    return _tc_apply(
        susum, tar_feat, W1.T, b1.reshape(1, D), W2.T, b2.reshape(1, D))


# speedup vs baseline: 1.0707x; 1.0707x over previous
import functools

import jax
import jax.numpy as jnp
from jax import lax
from jax.experimental import pallas as pl
from jax.experimental.pallas import tpu as pltpu
from jax.experimental.pallas import tpu_sc as plsc

N = 100000
M = 50000
D = 128

NC = 2
NS = 16
L = 16
NW = NC * NS
RANGE = 3200
N_PAD = NW * RANGE
CH = 2000
GCH = 128
NCG = RANGE // GCH
BLK = 4000
Z = 4096
ZBASE = 2 * M

_mesh = plsc.VectorSubcoreMesh(
    core_axis_name="c", subcore_axis_name="s", num_cores=NC, num_subcores=NS
)
_sc_params = pltpu.CompilerParams(
    needs_layout_passes=False, use_tc_tiling_on_sc=True)


@functools.partial(
    pl.kernel,
    out_type=(
        jax.ShapeDtypeStruct((NW, NCG, GCH), jnp.int32),
        jax.ShapeDtypeStruct((NW, NCG, GCH), jnp.int32),
    ),
    mesh=_mesh,
    compiler_params=_sc_params,
    scratch_types=[
        pltpu.VMEM((CH,), jnp.int32),
        pltpu.VMEM((CH,), jnp.int32),
        pltpu.VMEM((RANGE,), jnp.int32),
        pltpu.VMEM((NCG, GCH), jnp.int32),
        pltpu.SemaphoreType.DMA,
        pltpu.SemaphoreType.DMA,
    ],
)
def _sc_scan(o_idx, s_idx, gox, gsx, idx_a, idx_b, slot_v, gidx_v,
             isem_a, isem_b):
    cid = lax.axis_index("c")
    sid = lax.axis_index("s")
    wid = sid * NC + cid
    base = wid * RANGE
    lane = lax.iota(jnp.int32, L)

    GRP = 25
    NGRP = CH // L // GRP
    NCHUNK = M // CH

    def scan_stream(idx_hbm, g_hbm, soff):
        neg = jnp.full((L,), -1, jnp.int32)

        def init_body(v, carry):
            slot_v[pl.ds(v * L, L)] = neg
            return carry

        lax.fori_loop(0, RANGE // L, init_body, 0)

        pltpu.async_copy(idx_hbm.at[pl.ds(0, CH)], idx_a, isem_a)

        def process_half(c, idx_v):
            def grp_body(g, carry2):
                j0 = c * (CH // L) + g * GRP
                lostv = jnp.zeros((L,), jnp.int32)
                datas = []
                for u in range(GRP):
                    idxv = idx_v[pl.ds((g * GRP + u) * L, L)]
                    loc = idxv - base
                    m = (loc >= 0) & (loc < RANGE)
                    locm = jnp.where(m, loc, 0)
                    iv = (j0 + u) * L + lane
                    plsc.store_scatter(slot_v, (locm,), iv, mask=m)
                    datas.append((locm, iv, m))
                for locm, iv, m in datas:
                    back = plsc.load_gather(slot_v, (locm,), mask=m)
                    lostv = lostv | jnp.where(m & (back != iv), 1, 0)

                @pl.when(jnp.any(lostv > 0))
                def _repair():
                    def rep_body(u2, carry3):
                        idxv = idx_v[pl.ds((g * GRP + u2) * L, L)]
                        loc = idxv - base
                        m = (loc >= 0) & (loc < RANGE)
                        locm = jnp.where(m, loc, 0)
                        iv = (j0 + u2) * L + lane
                        for l in range(L):
                            plsc.store_scatter(
                                slot_v, (locm,), iv, mask=m & (lane == l))
                        return carry3

                    lax.fori_loop(0, GRP, rep_body, 0)

                return carry2

            lax.fori_loop(0, NGRP, grp_body, 0)

        def chunk_body(c, carry):
            @pl.when(c % 2 == 0)
            def _even():
                pltpu.make_async_copy(
                    idx_hbm.at[pl.ds(c * CH, CH)], idx_a, isem_a).wait()

                @pl.when(c + 1 < NCHUNK)
                def _pf():
                    pltpu.async_copy(
                        idx_hbm.at[pl.ds((c + 1) * CH, CH)], idx_b, isem_b)

                process_half(c, idx_a)

            @pl.when(c % 2 == 1)
            def _odd():
                pltpu.make_async_copy(
                    idx_hbm.at[pl.ds(c * CH, CH)], idx_b, isem_b).wait()

                @pl.when(c + 1 < NCHUNK)
                def _pf2():
                    pltpu.async_copy(
                        idx_hbm.at[pl.ds((c + 1) * CH, CH)], idx_a, isem_a)

                process_half(c, idx_b)

            return carry

        lax.fori_loop(0, NCHUNK, chunk_body, 0)

        def gidx_body(v, carry):
            sl = slot_v[pl.ds(v * L, L)]
            inval = sl < 0
            zfill = ZBASE + ((base + v * L + lane) & (Z - 1))
            g = jnp.where(inval, zfill, sl + soff)
            gidx_v[v // 8, pl.ds((v % 8) * L, L)] = g
            return carry

        lax.fori_loop(0, RANGE // L, gidx_body, 0)
        pltpu.sync_copy(gidx_v, g_hbm.at[wid])

    scan_stream(o_idx, gox, 0)
    scan_stream(s_idx, gsx, M)


@functools.partial(
    pl.kernel,
    out_type=jax.ShapeDtypeStruct((N_PAD, D), jnp.float32),
    mesh=_mesh,
    compiler_params=_sc_params,
    scratch_types=[
        pltpu.VMEM((NCG, GCH), jnp.int32),
        pltpu.VMEM((NCG, GCH), jnp.int32),
        pltpu.VMEM((GCH, D), jnp.float32),
        pltpu.VMEM((GCH, D), jnp.float32),
        pltpu.VMEM((GCH, D), jnp.float32),
        pltpu.VMEM((GCH, D), jnp.float32),
        pltpu.VMEM((GCH, D), jnp.float32),
        pltpu.VMEM((GCH, D), jnp.float32),
        pltpu.SemaphoreType.DMA,
        pltpu.SemaphoreType.DMA,
        pltpu.SemaphoreType.DMA,
        pltpu.SemaphoreType.DMA,
        pltpu.SemaphoreType.DMA,
        pltpu.SemaphoreType.DMA,
        pltpu.SemaphoreType.DMA,
    ],
)
def _sc_gather(big, gox, gsx, susum,
               gidx_o, gidx_s, bo0, bo1, bs0, bs1, bu0, bu1,
               gsem_o0, gsem_o1, gsem_s0, gsem_s1, osem0, osem1, lsem):
    cid = lax.axis_index("c")
    sid = lax.axis_index("s")
    wid = sid * NC + cid
    base = wid * RANGE

    pltpu.async_copy(gox.at[wid], gidx_o, lsem).wait()
    pltpu.async_copy(gsx.at[wid], gidx_s, lsem).wait()

    obufs = (bo0, bo1)
    sbufs = (bs0, bs1)
    ubufs = (bu0, bu1)
    osems = (gsem_o0, gsem_o1)
    ssems = (gsem_s0, gsem_s1)
    usems = (osem0, osem1)

    in_o = [None] * NCG
    in_s = [None] * NCG
    outd = [None] * NCG

    def fire(c):
        b = c % 2
        in_o[c] = pltpu.async_copy(big.at[gidx_o.at[c]], obufs[b], osems[b])
        in_s[c] = pltpu.async_copy(big.at[gidx_s.at[c]], sbufs[b], ssems[b])

    def addsum(b):
        def ab(k, carry):
            bu = ubufs[b]
            bo = obufs[b]
            bs = sbufs[b]
            for u in range(D // L):
                cs = pl.ds(u * L, L)
                bu[k, cs] = bo[k, cs] + bs[k, cs]
            return carry

        lax.fori_loop(0, GCH, ab, 0)

    fire(0)
    for c in range(NCG):
        b = c % 2
        if c + 1 < NCG:
            fire(c + 1)
        in_o[c].wait()
        in_s[c].wait()
        if c >= 2:
            outd[c - 2].wait()
        addsum(b)
        outd[c] = pltpu.async_copy(
            ubufs[b], susum.at[pl.ds(base + c * GCH, GCH)], usems[b])
    outd[NCG - 2].wait()
    outd[NCG - 1].wait()


def _tc_body(u_ref, tar_ref, w1_ref, b1_ref, w2_ref, b2_ref, out_ref):
    su = 0.5 * u_ref[...]
    h1 = jnp.maximum(
        jnp.dot(su, w1_ref[...], preferred_element_type=jnp.float32)
        + b1_ref[...], 0.0)
    h2 = jnp.maximum(
        jnp.dot(tar_ref[...], w2_ref[...], preferred_element_type=jnp.float32)
        + b2_ref[...], 0.0)
    out_ref[...] = su + h1 + h2


_tc_apply = pl.pallas_call(
    _tc_body,
    grid=(N // BLK,),
    in_specs=[
        pl.BlockSpec((BLK, D), lambda i: (i, 0)),
        pl.BlockSpec((BLK, D), lambda i: (i, 0)),
        pl.BlockSpec((D, D), lambda i: (0, 0)),
        pl.BlockSpec((1, D), lambda i: (0, 0)),
        pl.BlockSpec((D, D), lambda i: (0, 0)),
        pl.BlockSpec((1, D), lambda i: (0, 0)),
    ],
    out_specs=pl.BlockSpec((BLK, D), lambda i: (i, 0)),
    out_shape=jax.ShapeDtypeStruct((N, D), jnp.float32),
)


def kernel(msg_from_o, msg_from_s, o_ava_idx, s_ava_idx, tar_feat,
           W1, b1, W2, b2):
    big = jnp.concatenate(
        [msg_from_o, msg_from_s, jnp.zeros((Z, D), jnp.float32)], axis=0)
    gox, gsx = _sc_scan(
        o_ava_idx.astype(jnp.int32), s_ava_idx.astype(jnp.int32))
    susum = _sc_gather(big, gox, gsx)
    return _tc_apply(
        susum, tar_feat, W1.T, b1.reshape(1, D), W2.T, b2.reshape(1, D))
